# baseline (speedup 1.0000x reference)
import functools

import jax
import jax.numpy as jnp
from jax import lax
from jax.experimental import pallas as pl
from jax.experimental.pallas import tpu as pltpu
from jax.experimental.pallas import tpu_sc as plsc

BAG = 64
CLS = 128
N = 16384
NC = 2
NS = 16
NW = NC * NS
HB = 128

RSC = 2048
RWS = RSC // NW
NTC = N - RSC
BM = 2048
GRID = NTC // BM


def _sc_body(yt_hbm, yp_hbm, sums_hbm, cnts_hbm,
             idx_v, rows_v, hist_v, zbuf_v, hall_v, acc_sh, hist_sh):
    c = lax.axis_index("c")
    s = lax.axis_index("s")
    wid = s * NC + c

    pltpu.sync_copy(yt_hbm.at[wid], idx_v)
    pltpu.sync_copy(yp_hbm.at[pl.ds(NTC + wid * RWS, RWS)], rows_v)

    z16 = jnp.zeros((16,), jnp.float32)
    zrow = BAG // NS
    for i in range(zrow):
        for j in range(CLS // 16):
            zbuf_v[i, pl.ds(j * 16, 16)] = z16
    pltpu.sync_copy(zbuf_v, acc_sh.at[pl.ds(s * zrow, zrow)])

    ones = jnp.ones((16,), jnp.float32)
    for j in range(HB // 16):
        hist_v[pl.ds(j * 16, 16)] = z16
    for q in range(RWS // 16):
        ids16 = idx_v[0, pl.ds(q * 16, 16)]
        plsc.addupdate_scatter(hist_v, [ids16], ones)

    plsc.subcore_barrier()

    pltpu.sync_copy(rows_v, acc_sh.at[idx_v.at[0]], add=True)
    pltpu.sync_copy(hist_v, hist_sh.at[s])

    plsc.subcore_barrier()

    @pl.when(s == 0)
    def _():
        pltpu.sync_copy(acc_sh, sums_hbm.at[c])
        pltpu.sync_copy(hist_sh, hall_v)
        for q in range(HB // 16):
            acc16 = hall_v[0, pl.ds(q * 16, 16)]
            for r in range(1, NS):
                acc16 = acc16 + hall_v[r, pl.ds(q * 16, 16)]
            hist_v[pl.ds(q * 16, 16)] = acc16
        pltpu.sync_copy(hist_v, cnts_hbm.at[c])


def _sc_call(yt_sc, y_pred):
    call = functools.partial(
        pl.kernel,
        out_type=[
            jax.ShapeDtypeStruct((NC, BAG, CLS), jnp.float32),
            jax.ShapeDtypeStruct((NC, HB), jnp.float32),
        ],
        mesh=plsc.VectorSubcoreMesh(core_axis_name="c", subcore_axis_name="s",
                                    num_cores=NC, num_subcores=NS),
        scratch_types=[
            pltpu.VMEM((1, RWS), jnp.int32),
            pltpu.VMEM((RWS, CLS), jnp.float32),
            pltpu.VMEM((HB,), jnp.float32),
            pltpu.VMEM((BAG // NS, CLS), jnp.float32),
            pltpu.VMEM((NS, HB), jnp.float32),
            pltpu.VMEM_SHARED((BAG, CLS), jnp.float32),
            pltpu.VMEM_SHARED((NS, HB), jnp.float32),
        ],
        compiler_params=pltpu.CompilerParams(needs_layout_passes=False),
    )(_sc_body)
    return call(yt_sc, y_pred)


def _tc_body(yt_ref, yp_ref, sums_ref, cnt_ref):
    ids = yt_ref[...]
    oh = (jax.lax.broadcasted_iota(jnp.int32, (BAG, 1), 0) == ids).astype(
        jnp.float32
    )
    sums_ref[...] = jax.lax.dot_general(
        oh, yp_ref[...], (((1,), (0,)), ((), ())),
        preferred_element_type=jnp.float32,
    )
    cnt_ref[...] = jnp.sum(oh, axis=1, keepdims=True)


def _ep_body(ts_ref, tc_ref, ss_ref, sc_ref, th_ref, out_ref):
    sums = ts_ref[...] + ss_ref[0] + ss_ref[1]
    counts = tc_ref[...] + sc_ref[0] + sc_ref[1]
    means = sums / counts
    theta_c = jnp.clip(th_ref[...], 1e-07, 1.0 - 1e-07)
    m = jnp.max(means, axis=-1, keepdims=True)
    e = jnp.exp(means - m)
    ssum = jnp.sum(e, axis=-1, keepdims=True)
    sm = e / ssum
    loss = -theta_c * jnp.log(sm + 1e-07)
    out_ref[0, 0] = jnp.sum(loss)


def kernel(y_true, y_pred, theta):
    yt = y_true.astype(jnp.int32)
    yt_tc = yt[:NTC].reshape(1, NTC)
    yt_sc = yt[NTC:].reshape(NW, 1, RWS)

    sc_sums, sc_cnts = _sc_call(yt_sc, y_pred)

    tc_sums, tc_cnts = pl.pallas_call(
        _tc_body,
        grid=(1,),
        out_shape=[
            jax.ShapeDtypeStruct((BAG, CLS), jnp.float32),
            jax.ShapeDtypeStruct((BAG, 1), jnp.float32),
        ],
        in_specs=[
            pl.BlockSpec((1, NTC), lambda i: (0, 0)),
            pl.BlockSpec((NTC, CLS), lambda i: (0, 0)),
        ],
        out_specs=[
            pl.BlockSpec((BAG, CLS), lambda i: (0, 0)),
            pl.BlockSpec((BAG, 1), lambda i: (0, 0)),
        ],
    )(yt_tc, y_pred)

    out = pl.pallas_call(
        _ep_body,
        out_shape=jax.ShapeDtypeStruct((1, 1), jnp.float32),
        in_specs=[
            pl.BlockSpec(memory_space=pltpu.VMEM),
            pl.BlockSpec(memory_space=pltpu.VMEM),
            pl.BlockSpec(memory_space=pltpu.VMEM),
            pl.BlockSpec(memory_space=pltpu.VMEM),
            pl.BlockSpec(memory_space=pltpu.VMEM),
        ],
        out_specs=pl.BlockSpec(memory_space=pltpu.SMEM),
    )(tc_sums, tc_cnts, sc_sums,
      sc_cnts[:, :BAG].reshape(NC, BAG, 1), theta.reshape(BAG, 1))
    return out[0, 0]


# speedup vs baseline: 1.1303x; 1.1303x over previous
import functools

import jax
import jax.numpy as jnp
from jax import lax
from jax.experimental import pallas as pl
from jax.experimental.pallas import tpu as pltpu
from jax.experimental.pallas import tpu_sc as plsc

BAG = 64
CLS = 128
N = 16384
NC = 2
NS = 16
NW = NC * NS

RSC = 1024
RWS = RSC // NW
NTC = N - RSC


def _sc_body(yt_hbm, yp_hbm, sums_hbm, idx_v, rows_v, zbuf_v, acc_sh):
    c = lax.axis_index("c")
    s = lax.axis_index("s")
    wid = s * NC + c

    pltpu.sync_copy(yt_hbm.at[pl.ds(NTC + wid * RWS, RWS)], idx_v)
    pltpu.sync_copy(yp_hbm.at[pl.ds(NTC + wid * RWS, RWS)], rows_v)

    z16 = jnp.zeros((16,), jnp.float32)
    zrow = BAG // NS
    for i in range(zrow):
        for j in range(CLS // 16):
            zbuf_v[i, pl.ds(j * 16, 16)] = z16
    pltpu.sync_copy(zbuf_v, acc_sh.at[pl.ds(s * zrow, zrow)])

    plsc.subcore_barrier()

    pltpu.sync_copy(rows_v, acc_sh.at[idx_v], add=True)

    plsc.subcore_barrier()

    @pl.when(s == 0)
    def _():
        pltpu.sync_copy(acc_sh, sums_hbm.at[c])


def _sc_call(y_true, y_pred):
    call = functools.partial(
        pl.kernel,
        out_type=jax.ShapeDtypeStruct((NC, BAG, CLS), jnp.float32),
        mesh=plsc.VectorSubcoreMesh(core_axis_name="c", subcore_axis_name="s",
                                    num_cores=NC, num_subcores=NS),
        scratch_types=[
            pltpu.VMEM((RWS,), jnp.int32),
            pltpu.VMEM((RWS, CLS), jnp.float32),
            pltpu.VMEM((BAG // NS, CLS), jnp.float32),
            pltpu.VMEM_SHARED((BAG, CLS), jnp.float32),
        ],
        compiler_params=pltpu.CompilerParams(needs_layout_passes=False),
    )(_sc_body)
    return call(y_true, y_pred)


def _tc_body(yt_ref, yp_ref, sums_ref, cnt_ref):
    ids = yt_ref[...]
    iota = jax.lax.broadcasted_iota(jnp.int32, (BAG, 1), 0)
    oh_all = (iota == ids).astype(jnp.float32)
    cnt_ref[...] = jnp.sum(oh_all, axis=1, keepdims=True)
    sums_ref[...] = jax.lax.dot_general(
        oh_all[:, :NTC], yp_ref[...], (((1,), (0,)), ((), ())),
        preferred_element_type=jnp.float32,
    )


def _ep_body(ts_ref, ss_ref, cnt_ref, th_ref, out_ref):
    sums = ts_ref[...] + ss_ref[0] + ss_ref[1]
    means = sums / cnt_ref[...]
    theta_c = jnp.clip(th_ref[...], 1e-07, 1.0 - 1e-07)
    m = jnp.max(means, axis=-1, keepdims=True)
    e = jnp.exp(means - m)
    ssum = jnp.sum(e, axis=-1, keepdims=True)
    sm = e / ssum
    loss = -theta_c * jnp.log(sm + 1e-07)
    out_ref[0, 0] = jnp.sum(loss)


def kernel(y_true, y_pred, theta):
    yt = y_true.astype(jnp.int32)

    sc_sums = _sc_call(yt, y_pred)

    tc_sums, counts = pl.pallas_call(
        _tc_body,
        grid=(1,),
        out_shape=[
            jax.ShapeDtypeStruct((BAG, CLS), jnp.float32),
            jax.ShapeDtypeStruct((BAG, 1), jnp.float32),
        ],
        in_specs=[
            pl.BlockSpec((1, N), lambda i: (0, 0)),
            pl.BlockSpec((NTC, CLS), lambda i: (0, 0)),
        ],
        out_specs=[
            pl.BlockSpec((BAG, CLS), lambda i: (0, 0)),
            pl.BlockSpec((BAG, 1), lambda i: (0, 0)),
        ],
    )(yt.reshape(1, N), y_pred)

    out = pl.pallas_call(
        _ep_body,
        out_shape=jax.ShapeDtypeStruct((1, 1), jnp.float32),
        in_specs=[
            pl.BlockSpec(memory_space=pltpu.VMEM),
            pl.BlockSpec(memory_space=pltpu.VMEM),
            pl.BlockSpec(memory_space=pltpu.VMEM),
            pl.BlockSpec(memory_space=pltpu.VMEM),
        ],
        out_specs=pl.BlockSpec(memory_space=pltpu.SMEM),
    )(tc_sums, sc_sums, counts, theta.reshape(BAG, 1))
    return out[0, 0]
